# baseline (speedup 1.0000x reference)
import functools

import jax
import jax.numpy as jnp
from jax import lax
from jax.experimental import pallas as pl
from jax.experimental.pallas import tpu as pltpu
from jax.experimental.pallas import tpu_sc as plsc

N = 10000
E = 320000
H = 128
L = 3
D_IN = 196
D_EDGE = 7

NC = 2
NS = 16
NW = NC * NS
WIDE = 2 * H

CHUNK_G = 64
E_PAD = 327680
EW = E_PAD // NW
NPAIR_G = EW // (2 * CHUNK_G)
IDXROWS = EW // 128

CHUNK_S = 128
EW_S = E_PAD // NS
NCHUNKS_S = EW_S // CHUNK_S
N_ACC = 10112
ROWS_PER_TILE = N_ACC // NS

BE = 1024
BN = 1000


def _silu(v):
    return v * jax.nn.sigmoid(v)


_sc_mesh = plsc.VectorSubcoreMesh(core_axis_name="c", subcore_axis_name="s")


@functools.partial(
    pl.kernel,
    out_type=[
        jax.ShapeDtypeStruct((E_PAD, WIDE), jnp.float32),
        jax.ShapeDtypeStruct((E_PAD, WIDE), jnp.float32),
    ],
    mesh=_sc_mesh,
    scratch_types=[
        pltpu.VMEM((IDXROWS, 128), jnp.int32),
        pltpu.VMEM((IDXROWS, 128), jnp.int32),
        pltpu.VMEM((CHUNK_G, WIDE), jnp.float32),
        pltpu.VMEM((CHUNK_G, WIDE), jnp.float32),
        pltpu.VMEM((CHUNK_G, WIDE), jnp.float32),
        pltpu.VMEM((CHUNK_G, WIDE), jnp.float32),
        pltpu.SemaphoreType.DMA,
        pltpu.SemaphoreType.DMA,
    ],
)
def _sc_gather(a2_hbm, b2_hbm, src_hbm, dst_hbm, outa_hbm, outb_hbm,
               sidx, didx, bufa0, bufb0, bufa1, bufb1, sem0, sem1):
    wid = lax.axis_index("s") * NC + lax.axis_index("c")
    base = pl.multiple_of(wid * EW, 512)
    row0 = pl.multiple_of(wid * IDXROWS, 16)
    pltpu.sync_copy(src_hbm.at[pl.ds(row0, IDXROWS)], sidx)
    pltpu.sync_copy(dst_hbm.at[pl.ds(row0, IDXROWS)], didx)

    def fire(j, sub, bufa, bufb, sem):
        r = lax.rem(j, NPAIR_G)
        s = sidx.at[r, pl.ds(sub * CHUNK_G, CHUNK_G)]
        d = didx.at[r, pl.ds(sub * CHUNK_G, CHUNK_G)]
        pltpu.async_copy(a2_hbm.at[s], bufa, sem)
        pltpu.async_copy(b2_hbm.at[d], bufb, sem)

    def drain(j, sub, bufa, bufb, sem):
        r = lax.rem(j, NPAIR_G)
        s = sidx.at[r, pl.ds(sub * CHUNK_G, CHUNK_G)]
        d = didx.at[r, pl.ds(sub * CHUNK_G, CHUNK_G)]
        pltpu.make_async_copy(a2_hbm.at[s], bufa, sem).wait()
        pltpu.make_async_copy(b2_hbm.at[d], bufb, sem).wait()

    fire(0, 0, bufa0, bufb0, sem0)
    fire(0, 1, bufa1, bufb1, sem1)

    def body(j, _):
        for sub, (bufa, bufb, sem) in enumerate(
                ((bufa0, bufb0, sem0), (bufa1, bufb1, sem1))):
            off = pl.multiple_of(base + (2 * j + sub) * CHUNK_G, CHUNK_G)
            drain(j, sub, bufa, bufb, sem)
            pltpu.sync_copy(bufa, outa_hbm.at[pl.ds(off, CHUNK_G)])
            pltpu.sync_copy(bufb, outb_hbm.at[pl.ds(off, CHUNK_G)])
            fire(j + 1, sub, bufa, bufb, sem)
        return 0

    lax.fori_loop(0, NPAIR_G, body, 0)
    drain(NPAIR_G, 0, bufa0, bufb0, sem0)
    drain(NPAIR_G, 1, bufa1, bufb1, sem1)


@functools.partial(
    pl.kernel,
    out_type=jax.ShapeDtypeStruct((2 * N_ACC, H), jnp.float32),
    mesh=_sc_mesh,
    scratch_types=[
        pltpu.VMEM((NCHUNKS_S, CHUNK_S), jnp.int32),
        pltpu.VMEM((CHUNK_S, H), jnp.float32),
        pltpu.VMEM_SHARED((N_ACC, H), jnp.float32),
        pltpu.SemaphoreType.DMA,
    ],
)
def _sc_scatter(mx_hbm, dst_hbm, zeros_hbm, out_hbm, didx, buf, acc, sem):
    cid = lax.axis_index("c")
    sid = lax.axis_index("s")
    r0 = pl.multiple_of(sid * ROWS_PER_TILE, 8)
    pltpu.sync_copy(zeros_hbm.at[pl.ds(r0, ROWS_PER_TILE)],
                    acc.at[pl.ds(r0, ROWS_PER_TILE)])

    base = pl.multiple_of(sid * EW_S, 1024)
    idxrow0 = pl.multiple_of(sid * NCHUNKS_S, 32)
    pltpu.sync_copy(dst_hbm.at[pl.ds(idxrow0, NCHUNKS_S)], didx)
    plsc.subcore_barrier()
    col0 = pl.multiple_of(cid * H, H)

    def fire(i):
        r = lax.rem(i, NCHUNKS_S)
        off = pl.multiple_of(base + r * CHUNK_S, CHUNK_S)
        pltpu.async_copy(
            mx_hbm.at[pl.ds(off, CHUNK_S), pl.ds(col0, H)],
            buf, sem)

    def drain():
        pltpu.make_async_copy(
            mx_hbm.at[pl.ds(base, CHUNK_S), pl.ds(col0, H)], buf, sem).wait()

    fire(0)

    def body(i, _):
        drain()
        pltpu.sync_copy(buf, acc.at[didx.at[i]], add=True)
        fire(i + 1)
        return 0

    lax.fori_loop(0, NCHUNKS_S, body, 0)
    drain()
    plsc.subcore_barrier()
    outrow = pl.multiple_of(cid * N_ACC + r0, 8)
    pltpu.sync_copy(acc.at[pl.ds(r0, ROWS_PER_TILE)],
                    out_hbm.at[pl.ds(outrow, ROWS_PER_TILE)])


def _mk_table(feat, c8):
    pad = jnp.zeros((feat.shape[0], WIDE - H - 8), jnp.float32)
    return jnp.concatenate([feat, c8, pad], axis=1)


def _init_body(x_ref, c8_ref, wp_ref, bp_ref, ws_ref, wd_ref,
               h_ref, a2_ref, b2_ref):
    h = jnp.dot(x_ref[...], wp_ref[...], preferred_element_type=jnp.float32)
    h = h + bp_ref[...]
    c8 = c8_ref[...]
    a = jnp.dot(h, ws_ref[...], preferred_element_type=jnp.float32)
    b = jnp.dot(h, wd_ref[...], preferred_element_type=jnp.float32)
    h_ref[...] = h
    a2_ref[...] = _mk_table(a, c8)
    b2_ref[...] = _mk_table(b, -c8)


def _tc_init(x, c8, wp, bp, ws, wd):
    grid = (N // BN,)
    row = lambda i: (i, 0)
    fixed = lambda i: (0, 0)
    return pl.pallas_call(
        _init_body,
        grid=grid,
        in_specs=[
            pl.BlockSpec((BN, D_IN), row),
            pl.BlockSpec((BN, 8), row),
            pl.BlockSpec((D_IN, H), fixed),
            pl.BlockSpec((1, H), fixed),
            pl.BlockSpec((H, H), fixed),
            pl.BlockSpec((H, H), fixed),
        ],
        out_specs=[
            pl.BlockSpec((BN, H), row),
            pl.BlockSpec((BN, WIDE), row),
            pl.BlockSpec((BN, WIDE), row),
        ],
        out_shape=[
            jax.ShapeDtypeStruct((N, H), jnp.float32),
            jax.ShapeDtypeStruct((N, WIDE), jnp.float32),
            jax.ShapeDtypeStruct((N, WIDE), jnp.float32),
        ],
    )(x, c8, wp, bp, ws, wd)


def _edge_body(hs_ref, hd_ref, ea_ref, wea_ref, be1_ref, we2_ref, be2_ref,
               wc1_ref, bc1_ref, wc2_ref, bc2_ref, out_ref):
    blk = pl.program_id(0)
    g = hs_ref[...] + hd_ref[...]
    gh = g[:, :H]
    xd = g[:, H:H + 8]
    sqd = jnp.sum(xd * xd, axis=1, keepdims=True)

    lane8 = lax.broadcasted_iota(jnp.int32, (BE, 8), 1)
    ea8 = jnp.where(lane8 == 7, sqd, ea_ref[...])
    bf = jnp.bfloat16

    def bdot(u, v):
        return jnp.dot(u.astype(bf), v.astype(bf),
                       preferred_element_type=jnp.float32)

    pre1 = gh + bdot(ea8, wea_ref[...])
    pre1 = pre1 + be1_ref[...]
    m1 = _silu(pre1)
    m = _silu(bdot(m1, we2_ref[...]) + be2_ref[...])
    t = _silu(bdot(m, wc1_ref[...]) + bc1_ref[...])
    w8 = jnp.tanh(bdot(t, wc2_ref[...])
                  + bc2_ref[...])
    wcol = w8[:, 0:1]
    aux = xd * wcol
    aux = jnp.where(lane8 == 3, 1.0, aux)

    rid = blk * BE + lax.broadcasted_iota(jnp.int32, (BE, 1), 0)
    valid = rid < E
    m = jnp.where(valid, m, 0.0)
    aux = jnp.where(valid, aux, 0.0)
    pad = jnp.zeros((BE, WIDE - H - 8), jnp.float32)
    out_ref[...] = jnp.concatenate([m, aux, pad], axis=1)


def _tc_edge(hs, hd, ea, wea, be1, we2, be2, wc1, bc1, wc2p, bc2p):
    grid = (E_PAD // BE,)
    row = lambda i: (i, 0)
    fixed = lambda i: (0, 0)
    return pl.pallas_call(
        _edge_body,
        grid=grid,
        in_specs=[
            pl.BlockSpec((BE, WIDE), row),
            pl.BlockSpec((BE, WIDE), row),
            pl.BlockSpec((BE, 8), row),
            pl.BlockSpec((8, H), fixed),
            pl.BlockSpec((1, H), fixed),
            pl.BlockSpec((H, H), fixed),
            pl.BlockSpec((1, H), fixed),
            pl.BlockSpec((H, H), fixed),
            pl.BlockSpec((1, H), fixed),
            pl.BlockSpec((H, 8), fixed),
            pl.BlockSpec((1, 8), fixed),
        ],
        out_specs=pl.BlockSpec((BE, WIDE), row),
        out_shape=jax.ShapeDtypeStruct((E_PAD, WIDE), jnp.float32),
    )(hs, hd, ea, wea, be1, we2, be2, wc1, bc1, wc2p, bc2p)


def _node_body_factory(want_next):
    def body(h_ref, c8_ref, accm_ref, acca_ref, wn1a_ref, wn1b_ref, bn1_ref,
             wn2_ref, bn2_ref, ws_ref, wd_ref, *outs):
        h = h_ref[...]
        c8 = c8_ref[...]
        aggm = accm_ref[...]
        aux = acca_ref[...][:, :8]
        lane8 = lax.broadcasted_iota(jnp.int32, (BN, 8), 1)
        deg = jnp.maximum(aux[:, 3:4], 1.0)
        c_new = c8 + jnp.where(lane8 < 3, aux / deg, 0.0)
        hid = _silu(jnp.dot(h, wn1a_ref[...], preferred_element_type=jnp.float32)
                    + jnp.dot(aggm, wn1b_ref[...], preferred_element_type=jnp.float32)
                    + bn1_ref[...])
        h_new = h + jnp.dot(hid, wn2_ref[...], preferred_element_type=jnp.float32)
        h_new = h_new + bn2_ref[...]
        outs[0][...] = h_new
        outs[1][...] = c_new
        if want_next:
            a = jnp.dot(h_new, ws_ref[...], preferred_element_type=jnp.float32)
            b = jnp.dot(h_new, wd_ref[...], preferred_element_type=jnp.float32)
            outs[2][...] = _mk_table(a, c_new)
            outs[3][...] = _mk_table(b, -c_new)
    return body


def _tc_node(h, c8, accm, acca, wn1a, wn1b, bn1, wn2, bn2, ws, wd, want_next):
    grid = (N // BN,)
    row = lambda i: (i, 0)
    fixed = lambda i: (0, 0)
    out_specs = [pl.BlockSpec((BN, H), row), pl.BlockSpec((BN, 8), row)]
    out_shape = [
        jax.ShapeDtypeStruct((N, H), jnp.float32),
        jax.ShapeDtypeStruct((N, 8), jnp.float32),
    ]
    if want_next:
        out_specs += [pl.BlockSpec((BN, WIDE), row), pl.BlockSpec((BN, WIDE), row)]
        out_shape += [
            jax.ShapeDtypeStruct((N, WIDE), jnp.float32),
            jax.ShapeDtypeStruct((N, WIDE), jnp.float32),
        ]
    return pl.pallas_call(
        _node_body_factory(want_next),
        grid=grid,
        in_specs=[
            pl.BlockSpec((BN, H), row),
            pl.BlockSpec((BN, 8), row),
            pl.BlockSpec((BN, H), row),
            pl.BlockSpec((BN, H), row),
            pl.BlockSpec((H, H), fixed),
            pl.BlockSpec((H, H), fixed),
            pl.BlockSpec((1, H), fixed),
            pl.BlockSpec((H, H), fixed),
            pl.BlockSpec((1, H), fixed),
            pl.BlockSpec((H, H), fixed),
            pl.BlockSpec((H, H), fixed),
        ],
        out_specs=out_specs,
        out_shape=out_shape,
    )(h, c8, accm, acca, wn1a, wn1b, bn1, wn2, bn2, ws, wd)


@jax.jit
def _run(x, coords, edge_index, edge_attr, Wp, bp, We1, be1, We2, be2,
         Wc1, bc1, Wc2, bc2, Wn1, bn1, Wn2, bn2):
    f32 = jnp.float32
    src = edge_index[0].astype(jnp.int32)
    dst = edge_index[1].astype(jnp.int32)
    pad_e = E_PAD - E
    src_p = jnp.concatenate([src, jnp.zeros((pad_e,), jnp.int32)])
    dst_p = jnp.concatenate([dst, jnp.zeros((pad_e,), jnp.int32)])
    src2 = src_p.reshape(E_PAD // 128, 128)
    dst2 = dst_p.reshape(E_PAD // 128, 128)
    ea_p = jnp.zeros((E_PAD, 8), f32).at[:E, :D_EDGE].set(edge_attr)
    c8 = jnp.zeros((N, 8), f32).at[:, :3].set(coords)
    zeros_acc = jnp.zeros((N_ACC, H), f32)

    ws_l = We1[:, :H, :]
    wd_l = We1[:, H:2 * H, :]
    wea_l = jnp.concatenate([We1[:, 2 * H + 1:, :], We1[:, 2 * H:2 * H + 1, :]],
                            axis=1)
    be1_l = be1[:, None, :]
    be2_l = be2[:, None, :]
    bc1_l = bc1[:, None, :]
    wc2p_l = jnp.zeros((L, H, 8), f32).at[:, :, :1].set(Wc2)
    bc2p_l = jnp.zeros((L, 1, 8), f32).at[:, 0, 0].set(bc2[:, 0])
    wn1a_l = Wn1[:, :H, :]
    wn1b_l = Wn1[:, H:, :]
    bn1_l = bn1[:, None, :]
    bn2_l = bn2[:, None, :]
    bp2 = bp[None, :]

    h, a2, b2 = _tc_init(x, c8, Wp, bp2, ws_l[0], wd_l[0])
    for l in range(L):
        hs, hd = _sc_gather(a2, b2, src2, dst2)
        mx = _tc_edge(hs, hd, ea_p, wea_l[l], be1_l[l], We2[l], be2_l[l],
                      Wc1[l], bc1_l[l], wc2p_l[l], bc2p_l[l])
        acc = _sc_scatter(mx, dst2, zeros_acc)
        accm = lax.slice(acc, (0, 0), (N, H))
        acca = lax.slice(acc, (N_ACC, 0), (N_ACC + N, H))
        nxt = min(l + 1, L - 1)
        outs = _tc_node(h, c8, accm, acca, wn1a_l[l], wn1b_l[l], bn1_l[l], Wn2[l],
                        bn2_l[l], ws_l[nxt], wd_l[nxt], want_next=(l < L - 1))
        if l < L - 1:
            h, c8, a2, b2 = outs
        else:
            h, c8 = outs
    return jnp.concatenate([h, c8[:, :3]], axis=1)


def kernel(x, coords, edge_index, edge_attr, Wp, bp, We1, be1, We2, be2,
           Wc1, bc1, Wc2, bc2, Wn1, bn1, Wn2, bn2):
    return _run(x, coords, edge_index, edge_attr, Wp, bp, We1, be1, We2, be2,
                Wc1, bc1, Wc2, bc2, Wn1, bn1, Wn2, bn2)


# speedup vs baseline: 1.1592x; 1.1592x over previous
import functools

import jax
import jax.numpy as jnp
from jax import lax
from jax.experimental import pallas as pl
from jax.experimental.pallas import tpu as pltpu
from jax.experimental.pallas import tpu_sc as plsc

N = 10000
E = 320000
H = 128
L = 3
D_IN = 196
D_EDGE = 7

NC = 2
NS = 16
NW = NC * NS
WIDE = 2 * H

CHUNK_G = 64
E_PAD = 327680
EW = E_PAD // NW
NPAIR_G = EW // (2 * CHUNK_G)
IDXROWS = EW // 128

CHUNK_S = 128
EW_S = E_PAD // NS
NCHUNKS_S = EW_S // CHUNK_S
N_ACC = 10112
ROWS_PER_TILE = N_ACC // NS

BE = 2048
BN = 1000


def _silu(v):
    return v * jax.nn.sigmoid(v)


_sc_mesh = plsc.VectorSubcoreMesh(core_axis_name="c", subcore_axis_name="s")


@functools.partial(
    pl.kernel,
    out_type=[
        jax.ShapeDtypeStruct((E_PAD, WIDE), jnp.float32),
        jax.ShapeDtypeStruct((E_PAD, WIDE), jnp.float32),
    ],
    mesh=_sc_mesh,
    scratch_types=[
        pltpu.VMEM((IDXROWS, 128), jnp.int32),
        pltpu.VMEM((IDXROWS, 128), jnp.int32),
        pltpu.VMEM((CHUNK_G, WIDE), jnp.float32),
        pltpu.VMEM((CHUNK_G, WIDE), jnp.float32),
        pltpu.VMEM((CHUNK_G, WIDE), jnp.float32),
        pltpu.VMEM((CHUNK_G, WIDE), jnp.float32),
        pltpu.SemaphoreType.DMA,
        pltpu.SemaphoreType.DMA,
    ],
)
def _sc_gather(a2_hbm, b2_hbm, src_hbm, dst_hbm, outa_hbm, outb_hbm,
               sidx, didx, bufa0, bufb0, bufa1, bufb1, sem0, sem1):
    wid = lax.axis_index("s") * NC + lax.axis_index("c")
    base = pl.multiple_of(wid * EW, 512)
    row0 = pl.multiple_of(wid * IDXROWS, 16)
    pltpu.sync_copy(src_hbm.at[pl.ds(row0, IDXROWS)], sidx)
    pltpu.sync_copy(dst_hbm.at[pl.ds(row0, IDXROWS)], didx)

    def fire(j, sub, bufa, bufb, sem):
        r = lax.rem(j, NPAIR_G)
        s = sidx.at[r, pl.ds(sub * CHUNK_G, CHUNK_G)]
        d = didx.at[r, pl.ds(sub * CHUNK_G, CHUNK_G)]
        pltpu.async_copy(a2_hbm.at[s], bufa, sem)
        pltpu.async_copy(b2_hbm.at[d], bufb, sem)

    def drain(j, sub, bufa, bufb, sem):
        r = lax.rem(j, NPAIR_G)
        s = sidx.at[r, pl.ds(sub * CHUNK_G, CHUNK_G)]
        d = didx.at[r, pl.ds(sub * CHUNK_G, CHUNK_G)]
        pltpu.make_async_copy(a2_hbm.at[s], bufa, sem).wait()
        pltpu.make_async_copy(b2_hbm.at[d], bufb, sem).wait()

    fire(0, 0, bufa0, bufb0, sem0)
    fire(0, 1, bufa1, bufb1, sem1)

    def body(j, _):
        for sub, (bufa, bufb, sem) in enumerate(
                ((bufa0, bufb0, sem0), (bufa1, bufb1, sem1))):
            off = pl.multiple_of(base + (2 * j + sub) * CHUNK_G, CHUNK_G)
            drain(j, sub, bufa, bufb, sem)
            pltpu.sync_copy(bufa, outa_hbm.at[pl.ds(off, CHUNK_G)])
            pltpu.sync_copy(bufb, outb_hbm.at[pl.ds(off, CHUNK_G)])
            fire(j + 1, sub, bufa, bufb, sem)
        return 0

    lax.fori_loop(0, NPAIR_G, body, 0)
    drain(NPAIR_G, 0, bufa0, bufb0, sem0)
    drain(NPAIR_G, 1, bufa1, bufb1, sem1)


@functools.partial(
    pl.kernel,
    out_type=jax.ShapeDtypeStruct((2 * N_ACC, H), jnp.float32),
    mesh=_sc_mesh,
    scratch_types=[
        pltpu.VMEM((NCHUNKS_S // 2, CHUNK_S), jnp.int32),
        pltpu.VMEM((CHUNK_S, H), jnp.float32),
        pltpu.VMEM((CHUNK_S, H), jnp.float32),
        pltpu.VMEM_SHARED((N_ACC, H), jnp.float32),
        pltpu.SemaphoreType.DMA,
        pltpu.SemaphoreType.DMA,
    ],
)
def _sc_scatter(mx_hbm, dst_hbm, zeros_hbm, out_hbm, didx, buf0, buf1, acc,
                sem0, sem1):
    cid = lax.axis_index("c")
    sid = lax.axis_index("s")
    r0 = pl.multiple_of(sid * ROWS_PER_TILE, 8)
    pltpu.sync_copy(zeros_hbm.at[pl.ds(r0, ROWS_PER_TILE)],
                    acc.at[pl.ds(r0, ROWS_PER_TILE)])

    base = pl.multiple_of(sid * EW_S, 1024)
    half = NCHUNKS_S // 2
    idxrow0 = pl.multiple_of(sid * NCHUNKS_S, 32)
    pltpu.sync_copy(dst_hbm.at[pl.ds(idxrow0, half)], didx)
    plsc.subcore_barrier()
    col0 = pl.multiple_of(cid * H, H)

    def fire(i, buf, sem):
        r = lax.rem(i, NCHUNKS_S)
        off = pl.multiple_of(base + r * CHUNK_S, CHUNK_S)
        pltpu.async_copy(
            mx_hbm.at[pl.ds(off, CHUNK_S), pl.ds(col0, H)],
            buf, sem)

    def drain(buf, sem):
        pltpu.make_async_copy(
            mx_hbm.at[pl.ds(base, CHUNK_S), pl.ds(col0, H)], buf, sem).wait()

    fire(0, buf0, sem0)
    fire(1, buf1, sem1)

    def body(j, _):
        @pl.when(2 * j == half)
        def _():
            pltpu.sync_copy(
                dst_hbm.at[pl.ds(pl.multiple_of(idxrow0 + half, 16), half)],
                didx)

        for sub, (buf, sem) in enumerate(((buf0, sem0), (buf1, sem1))):
            i = 2 * j + sub
            drain(buf, sem)
            pltpu.sync_copy(buf, acc.at[didx.at[lax.rem(i, half)]], add=True)
            fire(i + 2, buf, sem)
        return 0

    lax.fori_loop(0, NCHUNKS_S // 2, body, 0)
    drain(buf0, sem0)
    drain(buf1, sem1)
    plsc.subcore_barrier()
    outrow = pl.multiple_of(cid * N_ACC + r0, 8)
    pltpu.sync_copy(acc.at[pl.ds(r0, ROWS_PER_TILE)],
                    out_hbm.at[pl.ds(outrow, ROWS_PER_TILE)])


def _mk_table(feat, c8):
    pad = jnp.zeros((feat.shape[0], WIDE - H - 8), jnp.float32)
    return jnp.concatenate([feat, c8, pad], axis=1)


def _init_body(x_ref, c8_ref, wp_ref, bp_ref, ws_ref, wd_ref,
               h_ref, a2_ref, b2_ref):
    h = jnp.dot(x_ref[...], wp_ref[...], preferred_element_type=jnp.float32)
    h = h + bp_ref[...]
    c8 = c8_ref[...]
    a = jnp.dot(h, ws_ref[...], preferred_element_type=jnp.float32)
    b = jnp.dot(h, wd_ref[...], preferred_element_type=jnp.float32)
    h_ref[...] = h
    a2_ref[...] = _mk_table(a, c8)
    b2_ref[...] = _mk_table(b, -c8)


def _tc_init(x, c8, wp, bp, ws, wd):
    grid = (N // BN,)
    row = lambda i: (i, 0)
    fixed = lambda i: (0, 0)
    return pl.pallas_call(
        _init_body,
        grid=grid,
        in_specs=[
            pl.BlockSpec((BN, D_IN), row),
            pl.BlockSpec((BN, 8), row),
            pl.BlockSpec((D_IN, H), fixed),
            pl.BlockSpec((1, H), fixed),
            pl.BlockSpec((H, H), fixed),
            pl.BlockSpec((H, H), fixed),
        ],
        out_specs=[
            pl.BlockSpec((BN, H), row),
            pl.BlockSpec((BN, WIDE), row),
            pl.BlockSpec((BN, WIDE), row),
        ],
        out_shape=[
            jax.ShapeDtypeStruct((N, H), jnp.float32),
            jax.ShapeDtypeStruct((N, WIDE), jnp.float32),
            jax.ShapeDtypeStruct((N, WIDE), jnp.float32),
        ],
    )(x, c8, wp, bp, ws, wd)


def _edge_body(hs_ref, hd_ref, ea_ref, wea_ref, be1_ref, we2_ref, be2_ref,
               wc1_ref, bc1_ref, wc2_ref, bc2_ref, out_ref):
    blk = pl.program_id(0)
    g = hs_ref[...] + hd_ref[...]
    gh = g[:, :H]
    xd = g[:, H:H + 8]
    sqd = jnp.sum(xd * xd, axis=1, keepdims=True)

    lane8 = lax.broadcasted_iota(jnp.int32, (BE, 8), 1)
    ea8 = jnp.where(lane8 == 7, sqd, ea_ref[...])
    bf = jnp.bfloat16

    def bdot(u, v):
        return jnp.dot(u.astype(bf), v.astype(bf),
                       preferred_element_type=jnp.float32)

    pre1 = gh + bdot(ea8, wea_ref[...])
    pre1 = pre1 + be1_ref[...]
    m1 = _silu(pre1)
    m = _silu(bdot(m1, we2_ref[...]) + be2_ref[...])
    t = _silu(bdot(m, wc1_ref[...]) + bc1_ref[...])
    w8 = jnp.tanh(bdot(t, wc2_ref[...])
                  + bc2_ref[...])
    wcol = w8[:, 0:1]
    aux = xd * wcol
    aux = jnp.where(lane8 == 3, 1.0, aux)

    rid = blk * BE + lax.broadcasted_iota(jnp.int32, (BE, 1), 0)
    valid = rid < E
    m = jnp.where(valid, m, 0.0)
    aux = jnp.where(valid, aux, 0.0)
    pad = jnp.zeros((BE, WIDE - H - 8), jnp.float32)
    out_ref[...] = jnp.concatenate([m, aux, pad], axis=1)


def _tc_edge(hs, hd, ea, wea, be1, we2, be2, wc1, bc1, wc2p, bc2p):
    grid = (E_PAD // BE,)
    row = lambda i: (i, 0)
    fixed = lambda i: (0, 0)
    return pl.pallas_call(
        _edge_body,
        grid=grid,
        in_specs=[
            pl.BlockSpec((BE, WIDE), row),
            pl.BlockSpec((BE, WIDE), row),
            pl.BlockSpec((BE, 8), row),
            pl.BlockSpec((8, H), fixed),
            pl.BlockSpec((1, H), fixed),
            pl.BlockSpec((H, H), fixed),
            pl.BlockSpec((1, H), fixed),
            pl.BlockSpec((H, H), fixed),
            pl.BlockSpec((1, H), fixed),
            pl.BlockSpec((H, 8), fixed),
            pl.BlockSpec((1, 8), fixed),
        ],
        out_specs=pl.BlockSpec((BE, WIDE), row),
        out_shape=jax.ShapeDtypeStruct((E_PAD, WIDE), jnp.float32),
    )(hs, hd, ea, wea, be1, we2, be2, wc1, bc1, wc2p, bc2p)


def _node_body_factory(want_next):
    def body(h_ref, c8_ref, accm_ref, acca_ref, wn1a_ref, wn1b_ref, bn1_ref,
             wn2_ref, bn2_ref, ws_ref, wd_ref, *outs):
        h = h_ref[...]
        c8 = c8_ref[...]
        aggm = accm_ref[...]
        aux = acca_ref[...][:, :8]
        lane8 = lax.broadcasted_iota(jnp.int32, (BN, 8), 1)
        deg = jnp.maximum(aux[:, 3:4], 1.0)
        c_new = c8 + jnp.where(lane8 < 3, aux / deg, 0.0)
        hid = _silu(jnp.dot(h, wn1a_ref[...], preferred_element_type=jnp.float32)
                    + jnp.dot(aggm, wn1b_ref[...], preferred_element_type=jnp.float32)
                    + bn1_ref[...])
        h_new = h + jnp.dot(hid, wn2_ref[...], preferred_element_type=jnp.float32)
        h_new = h_new + bn2_ref[...]
        outs[0][...] = h_new
        outs[1][...] = c_new
        if want_next:
            a = jnp.dot(h_new, ws_ref[...], preferred_element_type=jnp.float32)
            b = jnp.dot(h_new, wd_ref[...], preferred_element_type=jnp.float32)
            outs[2][...] = _mk_table(a, c_new)
            outs[3][...] = _mk_table(b, -c_new)
    return body


def _tc_node(h, c8, accm, acca, wn1a, wn1b, bn1, wn2, bn2, ws, wd, want_next):
    grid = (N // BN,)
    row = lambda i: (i, 0)
    fixed = lambda i: (0, 0)
    out_specs = [pl.BlockSpec((BN, H), row), pl.BlockSpec((BN, 8), row)]
    out_shape = [
        jax.ShapeDtypeStruct((N, H), jnp.float32),
        jax.ShapeDtypeStruct((N, 8), jnp.float32),
    ]
    if want_next:
        out_specs += [pl.BlockSpec((BN, WIDE), row), pl.BlockSpec((BN, WIDE), row)]
        out_shape += [
            jax.ShapeDtypeStruct((N, WIDE), jnp.float32),
            jax.ShapeDtypeStruct((N, WIDE), jnp.float32),
        ]
    return pl.pallas_call(
        _node_body_factory(want_next),
        grid=grid,
        in_specs=[
            pl.BlockSpec((BN, H), row),
            pl.BlockSpec((BN, 8), row),
            pl.BlockSpec((BN, H), row),
            pl.BlockSpec((BN, H), row),
            pl.BlockSpec((H, H), fixed),
            pl.BlockSpec((H, H), fixed),
            pl.BlockSpec((1, H), fixed),
            pl.BlockSpec((H, H), fixed),
            pl.BlockSpec((1, H), fixed),
            pl.BlockSpec((H, H), fixed),
            pl.BlockSpec((H, H), fixed),
        ],
        out_specs=out_specs,
        out_shape=out_shape,
    )(h, c8, accm, acca, wn1a, wn1b, bn1, wn2, bn2, ws, wd)


@jax.jit
def _run(x, coords, edge_index, edge_attr, Wp, bp, We1, be1, We2, be2,
         Wc1, bc1, Wc2, bc2, Wn1, bn1, Wn2, bn2):
    f32 = jnp.float32
    src = edge_index[0].astype(jnp.int32)
    dst = edge_index[1].astype(jnp.int32)
    pad_e = E_PAD - E
    src_p = jnp.concatenate([src, jnp.zeros((pad_e,), jnp.int32)])
    dst_p = jnp.concatenate([dst, jnp.zeros((pad_e,), jnp.int32)])
    src2 = src_p.reshape(E_PAD // 128, 128)
    dst2 = dst_p.reshape(E_PAD // 128, 128)
    ea_p = jnp.zeros((E_PAD, 8), f32).at[:E, :D_EDGE].set(edge_attr)
    c8 = jnp.zeros((N, 8), f32).at[:, :3].set(coords)
    zeros_acc = jnp.zeros((N_ACC, H), f32)

    ws_l = We1[:, :H, :]
    wd_l = We1[:, H:2 * H, :]
    wea_l = jnp.concatenate([We1[:, 2 * H + 1:, :], We1[:, 2 * H:2 * H + 1, :]],
                            axis=1)
    be1_l = be1[:, None, :]
    be2_l = be2[:, None, :]
    bc1_l = bc1[:, None, :]
    wc2p_l = jnp.zeros((L, H, 8), f32).at[:, :, :1].set(Wc2)
    bc2p_l = jnp.zeros((L, 1, 8), f32).at[:, 0, 0].set(bc2[:, 0])
    wn1a_l = Wn1[:, :H, :]
    wn1b_l = Wn1[:, H:, :]
    bn1_l = bn1[:, None, :]
    bn2_l = bn2[:, None, :]
    bp2 = bp[None, :]

    h, a2, b2 = _tc_init(x, c8, Wp, bp2, ws_l[0], wd_l[0])
    for l in range(L):
        hs, hd = _sc_gather(a2, b2, src2, dst2)
        mx = _tc_edge(hs, hd, ea_p, wea_l[l], be1_l[l], We2[l], be2_l[l],
                      Wc1[l], bc1_l[l], wc2p_l[l], bc2p_l[l])
        acc = _sc_scatter(mx, dst2, zeros_acc)
        accm = lax.slice(acc, (0, 0), (N, H))
        acca = lax.slice(acc, (N_ACC, 0), (N_ACC + N, H))
        nxt = min(l + 1, L - 1)
        outs = _tc_node(h, c8, accm, acca, wn1a_l[l], wn1b_l[l], bn1_l[l], Wn2[l],
                        bn2_l[l], ws_l[nxt], wd_l[nxt], want_next=(l < L - 1))
        if l < L - 1:
            h, c8, a2, b2 = outs
        else:
            h, c8 = outs
    return jnp.concatenate([h, c8[:, :3]], axis=1)


def kernel(x, coords, edge_index, edge_attr, Wp, bp, We1, be1, We2, be2,
           Wc1, bc1, Wc2, bc2, Wn1, bn1, Wn2, bn2):
    return _run(x, coords, edge_index, edge_attr, Wp, bp, We1, be1, We2, be2,
                Wc1, bc1, Wc2, bc2, Wn1, bn1, Wn2, bn2)


# speedup vs baseline: 1.2525x; 1.0805x over previous
import functools

import jax
import jax.numpy as jnp
from jax import lax
from jax.experimental import pallas as pl
from jax.experimental.pallas import tpu as pltpu
from jax.experimental.pallas import tpu_sc as plsc

N = 10000
E = 320000
H = 128
L = 3
D_IN = 196
D_EDGE = 7

NC = 2
NS = 16
NW = NC * NS
WIDE = 2 * H

CHUNK_G = 64
E_PAD = 327680
E_HALF = E_PAD // 2

CHUNK_S = 128
N_ACC = 10112
ROWS_PER_TILE = N_ACC // NS

BE = 2048
BN = 1000


def _silu(v):
    return v * jax.nn.sigmoid(v)


_sc_mesh = plsc.VectorSubcoreMesh(core_axis_name="c", subcore_axis_name="s")


def _make_gather(ebase, ecount):
    ew = ecount // NW
    idxrows = ew // 128
    npair = ew // (2 * CHUNK_G)

    @functools.partial(
        pl.kernel,
        out_type=[
            jax.ShapeDtypeStruct((ecount, WIDE), jnp.float32),
            jax.ShapeDtypeStruct((ecount, WIDE), jnp.float32),
        ],
        mesh=_sc_mesh,
        scratch_types=[
            pltpu.VMEM((idxrows, 128), jnp.int32),
            pltpu.VMEM((idxrows, 128), jnp.int32),
            pltpu.VMEM((CHUNK_G, WIDE), jnp.float32),
            pltpu.VMEM((CHUNK_G, WIDE), jnp.float32),
            pltpu.VMEM((CHUNK_G, WIDE), jnp.float32),
            pltpu.VMEM((CHUNK_G, WIDE), jnp.float32),
            pltpu.SemaphoreType.DMA,
            pltpu.SemaphoreType.DMA,
        ],
    )
    def gather(a2_hbm, b2_hbm, src_hbm, dst_hbm, outa_hbm, outb_hbm,
               sidx, didx, bufa0, bufb0, bufa1, bufb1, sem0, sem1):
        wid = lax.axis_index("s") * NC + lax.axis_index("c")
        base = pl.multiple_of(wid * ew, 512)
        row0 = pl.multiple_of(ebase // 128 + wid * idxrows, 8)
        pltpu.sync_copy(src_hbm.at[pl.ds(row0, idxrows)], sidx)
        pltpu.sync_copy(dst_hbm.at[pl.ds(row0, idxrows)], didx)

        def fire(j, sub, bufa, bufb, sem):
            r = lax.rem(j, npair)
            s = sidx.at[r, pl.ds(sub * CHUNK_G, CHUNK_G)]
            d = didx.at[r, pl.ds(sub * CHUNK_G, CHUNK_G)]
            pltpu.async_copy(a2_hbm.at[s], bufa, sem)
            pltpu.async_copy(b2_hbm.at[d], bufb, sem)

        def drain(j, sub, bufa, bufb, sem):
            r = lax.rem(j, npair)
            s = sidx.at[r, pl.ds(sub * CHUNK_G, CHUNK_G)]
            d = didx.at[r, pl.ds(sub * CHUNK_G, CHUNK_G)]
            pltpu.make_async_copy(a2_hbm.at[s], bufa, sem).wait()
            pltpu.make_async_copy(b2_hbm.at[d], bufb, sem).wait()

        fire(0, 0, bufa0, bufb0, sem0)
        fire(0, 1, bufa1, bufb1, sem1)

        def body(j, _):
            for sub, (bufa, bufb, sem) in enumerate(
                    ((bufa0, bufb0, sem0), (bufa1, bufb1, sem1))):
                off = pl.multiple_of(base + (2 * j + sub) * CHUNK_G, CHUNK_G)
                drain(j, sub, bufa, bufb, sem)
                pltpu.sync_copy(bufa, outa_hbm.at[pl.ds(off, CHUNK_G)])
                pltpu.sync_copy(bufb, outb_hbm.at[pl.ds(off, CHUNK_G)])
                fire(j + 1, sub, bufa, bufb, sem)
            return 0

        lax.fori_loop(0, npair, body, 0)
        drain(npair, 0, bufa0, bufb0, sem0)
        drain(npair, 1, bufa1, bufb1, sem1)

    return gather


_sc_gather_halves = (_make_gather(0, E_HALF), _make_gather(E_HALF, E_HALF))


def _make_scatter(ebase, ecount):
    ew_s = ecount // NS
    nchunks = ew_s // CHUNK_S

    @functools.partial(
        pl.kernel,
        out_type=jax.ShapeDtypeStruct((2 * N_ACC, H), jnp.float32),
        mesh=_sc_mesh,
        scratch_types=[
            pltpu.VMEM((nchunks, CHUNK_S), jnp.int32),
            pltpu.VMEM((CHUNK_S, H), jnp.float32),
            pltpu.VMEM((CHUNK_S, H), jnp.float32),
            pltpu.VMEM_SHARED((N_ACC, H), jnp.float32),
            pltpu.SemaphoreType.DMA,
            pltpu.SemaphoreType.DMA,
        ],
    )
    def scatter(mx_hbm, dst_hbm, zeros_hbm, out_hbm, didx, buf0, buf1, acc,
                sem0, sem1):
        cid = lax.axis_index("c")
        sid = lax.axis_index("s")
        r0 = pl.multiple_of(sid * ROWS_PER_TILE, 8)
        pltpu.sync_copy(zeros_hbm.at[pl.ds(r0, ROWS_PER_TILE)],
                        acc.at[pl.ds(r0, ROWS_PER_TILE)])

        base = pl.multiple_of(sid * ew_s, 1024)
        idxrow0 = pl.multiple_of(ebase // CHUNK_S + sid * nchunks, 8)
        pltpu.sync_copy(dst_hbm.at[pl.ds(idxrow0, nchunks)], didx)
        plsc.subcore_barrier()
        col0 = pl.multiple_of(cid * H, H)

        def fire(i, buf, sem):
            r = lax.rem(i, nchunks)
            off = pl.multiple_of(base + r * CHUNK_S, CHUNK_S)
            pltpu.async_copy(
                mx_hbm.at[pl.ds(off, CHUNK_S), pl.ds(col0, H)],
                buf, sem)

        def drain(buf, sem):
            pltpu.make_async_copy(
                mx_hbm.at[pl.ds(base, CHUNK_S), pl.ds(col0, H)],
                buf, sem).wait()

        fire(0, buf0, sem0)
        fire(1, buf1, sem1)

        def body(j, _):
            for sub, (buf, sem) in enumerate(((buf0, sem0), (buf1, sem1))):
                i = 2 * j + sub
                drain(buf, sem)
                pltpu.sync_copy(buf, acc.at[didx.at[i]], add=True)
                fire(i + 2, buf, sem)
            return 0

        lax.fori_loop(0, nchunks // 2, body, 0)
        drain(buf0, sem0)
        drain(buf1, sem1)
        plsc.subcore_barrier()
        outrow = pl.multiple_of(cid * N_ACC + r0, 8)
        pltpu.sync_copy(acc.at[pl.ds(r0, ROWS_PER_TILE)],
                        out_hbm.at[pl.ds(outrow, ROWS_PER_TILE)])

    return scatter


_sc_scatter_halves = (_make_scatter(0, E_HALF), _make_scatter(E_HALF, E_HALF))


def _mk_table(feat, c8):
    pad = jnp.zeros((feat.shape[0], WIDE - H - 8), jnp.float32)
    return jnp.concatenate([feat, c8, pad], axis=1)


def _init_body(x_ref, c8_ref, wp_ref, bp_ref, ws_ref, wd_ref,
               h_ref, a2_ref, b2_ref):
    h = jnp.dot(x_ref[...], wp_ref[...], preferred_element_type=jnp.float32)
    h = h + bp_ref[...]
    c8 = c8_ref[...]
    a = jnp.dot(h, ws_ref[...], preferred_element_type=jnp.float32)
    b = jnp.dot(h, wd_ref[...], preferred_element_type=jnp.float32)
    h_ref[...] = h
    a2_ref[...] = _mk_table(a, c8)
    b2_ref[...] = _mk_table(b, -c8)


def _tc_init(x, c8, wp, bp, ws, wd):
    grid = (N // BN,)
    row = lambda i: (i, 0)
    fixed = lambda i: (0, 0)
    return pl.pallas_call(
        _init_body,
        grid=grid,
        in_specs=[
            pl.BlockSpec((BN, D_IN), row),
            pl.BlockSpec((BN, 8), row),
            pl.BlockSpec((D_IN, H), fixed),
            pl.BlockSpec((1, H), fixed),
            pl.BlockSpec((H, H), fixed),
            pl.BlockSpec((H, H), fixed),
        ],
        out_specs=[
            pl.BlockSpec((BN, H), row),
            pl.BlockSpec((BN, WIDE), row),
            pl.BlockSpec((BN, WIDE), row),
        ],
        out_shape=[
            jax.ShapeDtypeStruct((N, H), jnp.float32),
            jax.ShapeDtypeStruct((N, WIDE), jnp.float32),
            jax.ShapeDtypeStruct((N, WIDE), jnp.float32),
        ],
    )(x, c8, wp, bp, ws, wd)


def _edge_body(ebase, hs_ref, hd_ref, ea_ref, wea_ref, be1_ref, we2_ref,
               be2_ref, wc1_ref, bc1_ref, wc2_ref, bc2_ref, out_ref):
    blk = pl.program_id(0)
    g = hs_ref[...] + hd_ref[...]
    gh = g[:, :H]
    xd = g[:, H:H + 8]
    sqd = jnp.sum(xd * xd, axis=1, keepdims=True)

    lane8 = lax.broadcasted_iota(jnp.int32, (BE, 8), 1)
    ea8 = jnp.where(lane8 == 7, sqd, ea_ref[...])
    bf = jnp.bfloat16

    def bdot(u, v):
        return jnp.dot(u.astype(bf), v.astype(bf),
                       preferred_element_type=jnp.float32)

    pre1 = gh + bdot(ea8, wea_ref[...])
    pre1 = pre1 + be1_ref[...]
    m1 = _silu(pre1)
    m = _silu(bdot(m1, we2_ref[...]) + be2_ref[...])
    t = _silu(bdot(m, wc1_ref[...]) + bc1_ref[...])
    w8 = jnp.tanh(bdot(t, wc2_ref[...])
                  + bc2_ref[...])
    wcol = w8[:, 0:1]
    aux = xd * wcol
    aux = jnp.where(lane8 == 3, 1.0, aux)

    rid = ebase + blk * BE + lax.broadcasted_iota(jnp.int32, (BE, 1), 0)
    valid = rid < E
    m = jnp.where(valid, m, 0.0)
    aux = jnp.where(valid, aux, 0.0)
    pad = jnp.zeros((BE, WIDE - H - 8), jnp.float32)
    out_ref[...] = jnp.concatenate([m, aux, pad], axis=1)


def _tc_edge(ebase, hs, hd, ea, wea, be1, we2, be2, wc1, bc1, wc2p, bc2p):
    grid = (E_HALF // BE,)
    row = lambda i: (i, 0)
    fixed = lambda i: (0, 0)
    return pl.pallas_call(
        functools.partial(_edge_body, ebase),
        grid=grid,
        in_specs=[
            pl.BlockSpec((BE, WIDE), row),
            pl.BlockSpec((BE, WIDE), row),
            pl.BlockSpec((BE, 8), row),
            pl.BlockSpec((8, H), fixed),
            pl.BlockSpec((1, H), fixed),
            pl.BlockSpec((H, H), fixed),
            pl.BlockSpec((1, H), fixed),
            pl.BlockSpec((H, H), fixed),
            pl.BlockSpec((1, H), fixed),
            pl.BlockSpec((H, 8), fixed),
            pl.BlockSpec((1, 8), fixed),
        ],
        out_specs=pl.BlockSpec((BE, WIDE), row),
        out_shape=jax.ShapeDtypeStruct((E_HALF, WIDE), jnp.float32),
    )(hs, hd, ea, wea, be1, we2, be2, wc1, bc1, wc2p, bc2p)


def _node_body_factory(want_next):
    def body(h_ref, c8_ref, accm1_ref, acca1_ref, accm2_ref, acca2_ref,
             wn1a_ref, wn1b_ref, bn1_ref,
             wn2_ref, bn2_ref, ws_ref, wd_ref, *outs):
        h = h_ref[...]
        c8 = c8_ref[...]
        aggm = accm1_ref[...] + accm2_ref[...]
        aux = (acca1_ref[...] + acca2_ref[...])[:, :8]
        lane8 = lax.broadcasted_iota(jnp.int32, (BN, 8), 1)
        deg = jnp.maximum(aux[:, 3:4], 1.0)
        c_new = c8 + jnp.where(lane8 < 3, aux / deg, 0.0)
        hid = _silu(jnp.dot(h, wn1a_ref[...], preferred_element_type=jnp.float32)
                    + jnp.dot(aggm, wn1b_ref[...], preferred_element_type=jnp.float32)
                    + bn1_ref[...])
        h_new = h + jnp.dot(hid, wn2_ref[...], preferred_element_type=jnp.float32)
        h_new = h_new + bn2_ref[...]
        outs[0][...] = h_new
        outs[1][...] = c_new
        if want_next:
            a = jnp.dot(h_new, ws_ref[...], preferred_element_type=jnp.float32)
            b = jnp.dot(h_new, wd_ref[...], preferred_element_type=jnp.float32)
            outs[2][...] = _mk_table(a, c_new)
            outs[3][...] = _mk_table(b, -c_new)
    return body


def _tc_node(h, c8, accs, wn1a, wn1b, bn1, wn2, bn2, ws, wd, want_next):
    grid = (N // BN,)
    row = lambda i: (i, 0)
    fixed = lambda i: (0, 0)
    out_specs = [pl.BlockSpec((BN, H), row), pl.BlockSpec((BN, 8), row)]
    out_shape = [
        jax.ShapeDtypeStruct((N, H), jnp.float32),
        jax.ShapeDtypeStruct((N, 8), jnp.float32),
    ]
    if want_next:
        out_specs += [pl.BlockSpec((BN, WIDE), row), pl.BlockSpec((BN, WIDE), row)]
        out_shape += [
            jax.ShapeDtypeStruct((N, WIDE), jnp.float32),
            jax.ShapeDtypeStruct((N, WIDE), jnp.float32),
        ]
    return pl.pallas_call(
        _node_body_factory(want_next),
        grid=grid,
        in_specs=[
            pl.BlockSpec((BN, H), row),
            pl.BlockSpec((BN, 8), row),
            pl.BlockSpec((BN, H), row),
            pl.BlockSpec((BN, H), row),
            pl.BlockSpec((BN, H), row),
            pl.BlockSpec((BN, H), row),
            pl.BlockSpec((H, H), fixed),
            pl.BlockSpec((H, H), fixed),
            pl.BlockSpec((1, H), fixed),
            pl.BlockSpec((H, H), fixed),
            pl.BlockSpec((1, H), fixed),
            pl.BlockSpec((H, H), fixed),
            pl.BlockSpec((H, H), fixed),
        ],
        out_specs=out_specs,
        out_shape=out_shape,
    )(h, c8, *accs, wn1a, wn1b, bn1, wn2, bn2, ws, wd)


@jax.jit
def _run(x, coords, edge_index, edge_attr, Wp, bp, We1, be1, We2, be2,
         Wc1, bc1, Wc2, bc2, Wn1, bn1, Wn2, bn2):
    f32 = jnp.float32
    src = edge_index[0].astype(jnp.int32)
    dst = edge_index[1].astype(jnp.int32)
    pad_e = E_PAD - E
    src_p = jnp.concatenate([src, jnp.zeros((pad_e,), jnp.int32)])
    dst_p = jnp.concatenate([dst, jnp.zeros((pad_e,), jnp.int32)])
    src2 = src_p.reshape(E_PAD // 128, 128)
    dst2 = dst_p.reshape(E_PAD // 128, 128)
    ea_p = jnp.zeros((E_PAD, 8), f32).at[:E, :D_EDGE].set(edge_attr)
    c8 = jnp.zeros((N, 8), f32).at[:, :3].set(coords)
    zeros_acc = jnp.zeros((N_ACC, H), f32)

    ws_l = We1[:, :H, :]
    wd_l = We1[:, H:2 * H, :]
    wea_l = jnp.concatenate([We1[:, 2 * H + 1:, :], We1[:, 2 * H:2 * H + 1, :]],
                            axis=1)
    be1_l = be1[:, None, :]
    be2_l = be2[:, None, :]
    bc1_l = bc1[:, None, :]
    wc2p_l = jnp.zeros((L, H, 8), f32).at[:, :, :1].set(Wc2)
    bc2p_l = jnp.zeros((L, 1, 8), f32).at[:, 0, 0].set(bc2[:, 0])
    wn1a_l = Wn1[:, :H, :]
    wn1b_l = Wn1[:, H:, :]
    bn1_l = bn1[:, None, :]
    bn2_l = bn2[:, None, :]
    bp2 = bp[None, :]

    ea_halves = (lax.slice(ea_p, (0, 0), (E_HALF, 8)),
                 lax.slice(ea_p, (E_HALF, 0), (E_PAD, 8)))

    h, a2, b2 = _tc_init(x, c8, Wp, bp2, ws_l[0], wd_l[0])
    for l in range(L):
        accs = []
        mxs = [None, None]
        hs0, hd0 = _sc_gather_halves[0](a2, b2, src2, dst2)
        mxs[0] = _tc_edge(0, hs0, hd0, ea_halves[0], wea_l[l], be1_l[l],
                          We2[l], be2_l[l], Wc1[l], bc1_l[l], wc2p_l[l],
                          bc2p_l[l])
        hs1, hd1 = _sc_gather_halves[1](a2, b2, src2, dst2)
        acc0 = _sc_scatter_halves[0](mxs[0], dst2, zeros_acc)
        mxs[1] = _tc_edge(E_HALF, hs1, hd1, ea_halves[1], wea_l[l], be1_l[l],
                          We2[l], be2_l[l], Wc1[l], bc1_l[l], wc2p_l[l],
                          bc2p_l[l])
        acc1 = _sc_scatter_halves[1](mxs[1], dst2, zeros_acc)
        for acc in (acc0, acc1):
            accs.append(lax.slice(acc, (0, 0), (N, H)))
            accs.append(lax.slice(acc, (N_ACC, 0), (N_ACC + N, H)))
        nxt = min(l + 1, L - 1)
        outs = _tc_node(h, c8, accs, wn1a_l[l], wn1b_l[l], bn1_l[l], Wn2[l],
                        bn2_l[l], ws_l[nxt], wd_l[nxt], want_next=(l < L - 1))
        if l < L - 1:
            h, c8, a2, b2 = outs
        else:
            h, c8 = outs
    return jnp.concatenate([h, c8[:, :3]], axis=1)


def kernel(x, coords, edge_index, edge_attr, Wp, bp, We1, be1, We2, be2,
           Wc1, bc1, Wc2, bc2, Wn1, bn1, Wn2, bn2):
    return _run(x, coords, edge_index, edge_attr, Wp, bp, We1, be1, We2, be2,
                Wc1, bc1, Wc2, bc2, Wn1, bn1, Wn2, bn2)
